# baseline (speedup 1.0000x reference)
import functools

import jax
import jax.numpy as jnp
from jax import lax
from jax.experimental import pallas as pl
from jax.experimental.pallas import tpu as pltpu
from jax.experimental.pallas import tpu_sc as plsc

NC = 2
NS = 16
LANES = 16

BATCH = 32
CHANNELS = 3
IMG = 512
PLANE = IMG * IMG
NBINS = 16
FEAT = CHANNELS * NBINS
PAD = 128
OUT_DIM = 64
UNROLL = 8

ROW_SC = 160
ROWS = 32
CPP = ROW_SC // ROWS
CHUNK = ROWS * IMG

RB = 32
K_TC = (IMG - ROW_SC) // RB
OFF_B = ROW_SC // RB


def _sc_body(x_hbm, out_hbm,
             buf0, buf1, hist_v, out_v, sem0, sem1):
  b = lax.axis_index("s") * NC + lax.axis_index("c")

  bufs = (buf0, buf1)
  sems = (sem0, sem1)
  total = CHANNELS * CPP

  def start_dma(t):
    c, ch = divmod(t, CPP)
    return pltpu.async_copy(
        x_hbm.at[b, c, pl.ds(ch * ROWS, ROWS), :], bufs[t % 2], sems[t % 2])

  pending = start_dma(0)

  lane = lax.iota(jnp.int32, LANES)
  ones = jnp.ones((LANES,), jnp.float32)
  zeros = jnp.zeros((LANES,), jnp.float32)
  for j in range(PAD // LANES):
    out_v[pl.ds(j * LANES, LANES)] = zeros

  for c in range(CHANNELS):
    for l in range(LANES):
      hist_v[l, :] = zeros
    for ch in range(CPP):
      t = c * CPP + ch
      nxt = start_dma(t + 1) if t + 1 < total else None
      pending.wait()
      buf = bufs[t % 2]

      @plsc.parallel_loop(0, CHUNK, step=LANES, unroll=UNROLL)
      def _(i, buf=buf):
        row = lax.shift_right_logical(i, 9)
        col = lax.bitwise_and(i, IMG - 1)
        v = buf[row, pl.ds(col, LANES)]
        bins = (v * 16.0).astype(jnp.int32)
        plsc.addupdate_scatter(hist_v, [lane, bins], ones)

      pending = nxt
    acc = hist_v[0, :]
    for l in range(1, LANES):
      acc = acc + hist_v[l, :]
    out_v[pl.ds(c * NBINS, LANES)] = acc
  pltpu.sync_copy(out_v, out_hbm.at[b])


def _sc_counts(x):
  mesh = plsc.VectorSubcoreMesh(core_axis_name="c", subcore_axis_name="s")
  fn = pl.kernel(
      _sc_body,
      out_type=jax.ShapeDtypeStruct((BATCH, PAD), jnp.float32),
      mesh=mesh,
      compiler_params=pltpu.CompilerParams(
          needs_layout_passes=False, use_tc_tiling_on_sc=True),
      scratch_types=[
          pltpu.VMEM((ROWS, IMG), jnp.float32),
          pltpu.VMEM((ROWS, IMG), jnp.float32),
          pltpu.VMEM((LANES, NBINS), jnp.float32),
          pltpu.VMEM((PAD,), jnp.float32),
          pltpu.SemaphoreType.DMA,
          pltpu.SemaphoreType.DMA,
      ],
  )
  return fn(x)


def _tc_hist_body(x_ref, out_ref):
  k = pl.program_id(1)
  rows = []
  for c in range(CHANNELS):
    xc = x_ref[0, c]
    rows.append(jnp.full((IMG,), jnp.float32(RB)))
    rows.extend(jnp.sum((xc >= (i / 16.0)).astype(jnp.float32), axis=0)
                for i in range(1, NBINS))
  blk = jnp.stack(rows).reshape(1, FEAT, IMG)

  @pl.when(k == 0)
  def _():
    out_ref[...] = blk

  @pl.when(k != 0)
  def _():
    out_ref[...] += blk


def _tc_counts(x):
  return pl.pallas_call(
      _tc_hist_body,
      grid=(BATCH, K_TC),
      in_specs=[pl.BlockSpec((1, CHANNELS, RB, IMG),
                             lambda i, k: (i, 0, k + OFF_B, 0))],
      out_specs=pl.BlockSpec((1, FEAT, IMG), lambda i, k: (i, 0, 0)),
      out_shape=jax.ShapeDtypeStruct((BATCH, FEAT, IMG), jnp.float32),
  )(x)


def _fc_body(a_ref, c_ref, wp_ref, w_ref, bias_ref, o_ref):
  tails = jnp.sum(c_ref[...], axis=2)
  nxt = jnp.concatenate(
      [tails[:, 1:], jnp.zeros((BATCH, 1), jnp.float32)], axis=1)
  mask = (lax.broadcasted_iota(jnp.int32, (1, FEAT), 1) % NBINS
          ) != (NBINS - 1)
  cnt_tc = tails - jnp.where(mask, nxt, 0.0)
  scale = jnp.float32(1.0 / PLANE)
  o = jnp.dot(a_ref[...] * scale, wp_ref[...],
              preferred_element_type=jnp.float32)
  o += jnp.dot(cnt_tc * scale, w_ref[...],
               preferred_element_type=jnp.float32)
  o_ref[...] = jnp.maximum(o + bias_ref[...][None, :], 0.0)


def _fc(cnt_sc, cnt_tc, w_pad, w, bias):
  return pl.pallas_call(
      _fc_body,
      out_shape=jax.ShapeDtypeStruct((BATCH, OUT_DIM), jnp.float32),
  )(cnt_sc, cnt_tc, w_pad, w, bias)


@jax.jit
def kernel(x, W, b):
  cnt_sc = _sc_counts(x)
  cnt_tc = _tc_counts(x)
  w_pad = jnp.zeros((PAD, OUT_DIM), jnp.float32).at[:FEAT].set(W)
  return _fc(cnt_sc, cnt_tc, w_pad, W, b)


# speedup vs baseline: 2.3007x; 2.3007x over previous
import functools

import jax
import jax.numpy as jnp
from jax import lax
from jax.experimental import pallas as pl
from jax.experimental.pallas import tpu as pltpu
from jax.experimental.pallas import tpu_sc as plsc

NC = 2
NS = 16
LANES = 16

BATCH = 32
CHANNELS = 3
IMG = 512
PLANE = IMG * IMG
NBINS = 16
FEAT = CHANNELS * NBINS
PAD = 128
OUT_DIM = 64
UNROLL = 8

ROW_SC = 384
ROWS = 32
CPP = ROW_SC // ROWS
CHUNK = ROWS * IMG

RB = 128
K_TC = (IMG - ROW_SC) // RB
OFF_B = ROW_SC // RB


def _sc_body(x_hbm, out_hbm,
             buf0, buf1, hist_v, out_v, sem0, sem1):
  b = lax.axis_index("s") * NC + lax.axis_index("c")

  bufs = (buf0, buf1)
  sems = (sem0, sem1)
  total = CHANNELS * CPP

  def start_dma(t):
    c, ch = divmod(t, CPP)
    return pltpu.async_copy(
        x_hbm.at[b, c, pl.ds(ch * ROWS, ROWS), :], bufs[t % 2], sems[t % 2])

  pending = start_dma(0)

  lane = lax.iota(jnp.int32, LANES)
  ones = jnp.ones((LANES,), jnp.float32)
  zeros = jnp.zeros((LANES,), jnp.float32)
  for j in range(PAD // LANES):
    out_v[pl.ds(j * LANES, LANES)] = zeros

  for c in range(CHANNELS):
    for l in range(LANES):
      hist_v[l, :] = zeros
    for ch in range(CPP):
      t = c * CPP + ch
      nxt = start_dma(t + 1) if t + 1 < total else None
      pending.wait()
      buf = bufs[t % 2]

      @plsc.parallel_loop(0, CHUNK, step=LANES, unroll=UNROLL)
      def _(i, buf=buf):
        row = lax.shift_right_logical(i, 9)
        col = lax.bitwise_and(i, IMG - 1)
        v = buf[row, pl.ds(col, LANES)]
        bins = (v * 16.0).astype(jnp.int32)
        plsc.addupdate_scatter(hist_v, [lane, bins], ones)

      pending = nxt
    acc = hist_v[0, :]
    for l in range(1, LANES):
      acc = acc + hist_v[l, :]
    out_v[pl.ds(c * NBINS, LANES)] = acc
  pltpu.sync_copy(out_v, out_hbm.at[b])


def _sc_counts(x):
  mesh = plsc.VectorSubcoreMesh(core_axis_name="c", subcore_axis_name="s")
  fn = pl.kernel(
      _sc_body,
      out_type=jax.ShapeDtypeStruct((BATCH, PAD), jnp.float32),
      mesh=mesh,
      compiler_params=pltpu.CompilerParams(
          needs_layout_passes=False, use_tc_tiling_on_sc=True),
      scratch_types=[
          pltpu.VMEM((ROWS, IMG), jnp.float32),
          pltpu.VMEM((ROWS, IMG), jnp.float32),
          pltpu.VMEM((LANES, NBINS), jnp.float32),
          pltpu.VMEM((PAD,), jnp.float32),
          pltpu.SemaphoreType.DMA,
          pltpu.SemaphoreType.DMA,
      ],
  )
  return fn(x)


def _tc_hist_body(x_ref, out_ref):
  k = pl.program_id(1)
  rows = []
  for c in range(CHANNELS):
    xc = x_ref[0, c]
    rows.append(jnp.full((IMG,), jnp.float32(RB)))
    rows.extend(jnp.sum((xc >= (i / 16.0)).astype(jnp.float32), axis=0)
                for i in range(1, NBINS))
  blk = jnp.stack(rows).reshape(1, FEAT, IMG)

  @pl.when(k == 0)
  def _():
    out_ref[...] = blk

  @pl.when(k != 0)
  def _():
    out_ref[...] += blk


def _tc_counts(x):
  return pl.pallas_call(
      _tc_hist_body,
      grid=(BATCH, K_TC),
      in_specs=[pl.BlockSpec((1, CHANNELS, RB, IMG),
                             lambda i, k: (i, 0, k + OFF_B, 0))],
      out_specs=pl.BlockSpec((1, FEAT, IMG), lambda i, k: (i, 0, 0)),
      out_shape=jax.ShapeDtypeStruct((BATCH, FEAT, IMG), jnp.float32),
  )(x)


def _fc_body(a_ref, c_ref, wp_ref, w_ref, bias_ref, o_ref):
  tails = jnp.sum(c_ref[...], axis=2)
  nxt = jnp.concatenate(
      [tails[:, 1:], jnp.zeros((BATCH, 1), jnp.float32)], axis=1)
  mask = (lax.broadcasted_iota(jnp.int32, (1, FEAT), 1) % NBINS
          ) != (NBINS - 1)
  cnt_tc = tails - jnp.where(mask, nxt, 0.0)
  scale = jnp.float32(1.0 / PLANE)
  o = jnp.dot(a_ref[...] * scale, wp_ref[...],
              preferred_element_type=jnp.float32)
  o += jnp.dot(cnt_tc * scale, w_ref[...],
               preferred_element_type=jnp.float32)
  o_ref[...] = jnp.maximum(o + bias_ref[...][None, :], 0.0)


def _fc(cnt_sc, cnt_tc, w_pad, w, bias):
  return pl.pallas_call(
      _fc_body,
      out_shape=jax.ShapeDtypeStruct((BATCH, OUT_DIM), jnp.float32),
  )(cnt_sc, cnt_tc, w_pad, w, bias)


@jax.jit
def kernel(x, W, b):
  cnt_sc = _sc_counts(x)
  cnt_tc = _tc_counts(x)
  w_pad = jnp.zeros((PAD, OUT_DIM), jnp.float32).at[:FEAT].set(W)
  return _fc(cnt_sc, cnt_tc, w_pad, W, b)


# speedup vs baseline: 2.5644x; 1.1146x over previous
import functools

import jax
import jax.numpy as jnp
from jax import lax
from jax.experimental import pallas as pl
from jax.experimental.pallas import tpu as pltpu
from jax.experimental.pallas import tpu_sc as plsc

NC = 2
NS = 16
LANES = 16

BATCH = 32
CHANNELS = 3
IMG = 512
PLANE = IMG * IMG
NBINS = 16
FEAT = CHANNELS * NBINS
PAD = 128
OUT_DIM = 64
UNROLL = 8

ROW_TC = 176
ROW_SC = IMG - ROW_TC
ROWS = 48
CPP = ROW_SC // ROWS
CHUNK = ROWS * IMG

RB = ROW_TC
K_TC = ROW_TC // RB
OFF_B = 0


def _sc_body(x_hbm, out_hbm,
             buf0, buf1, hist_v, out_v, sem0, sem1):
  b = lax.axis_index("s") * NC + lax.axis_index("c")

  bufs = (buf0, buf1)
  sems = (sem0, sem1)
  total = CHANNELS * CPP

  def start_dma(t):
    c, ch = divmod(t, CPP)
    return pltpu.async_copy(
        x_hbm.at[b, c, pl.ds(ROW_TC + ch * ROWS, ROWS), :],
        bufs[t % 2], sems[t % 2])

  pending = start_dma(0)

  lane = lax.iota(jnp.int32, LANES)
  ones = jnp.ones((LANES,), jnp.float32)
  zeros = jnp.zeros((LANES,), jnp.float32)
  for j in range(PAD // LANES):
    out_v[pl.ds(j * LANES, LANES)] = zeros

  for c in range(CHANNELS):
    for l in range(LANES):
      hist_v[l, :] = zeros
    for ch in range(CPP):
      t = c * CPP + ch
      nxt = start_dma(t + 1) if t + 1 < total else None
      pending.wait()
      buf = bufs[t % 2]

      @plsc.parallel_loop(0, CHUNK, step=LANES, unroll=UNROLL)
      def _(i, buf=buf):
        row = lax.shift_right_logical(i, 9)
        col = lax.bitwise_and(i, IMG - 1)
        v = buf[row, pl.ds(col, LANES)]
        bins = (v * 16.0).astype(jnp.int32)
        plsc.addupdate_scatter(hist_v, [lane, bins], ones)

      pending = nxt
    acc = hist_v[0, :]
    for l in range(1, LANES):
      acc = acc + hist_v[l, :]
    out_v[pl.ds(c * NBINS, LANES)] = acc
  pltpu.sync_copy(out_v, out_hbm.at[b])


def _sc_counts(x):
  mesh = plsc.VectorSubcoreMesh(core_axis_name="c", subcore_axis_name="s")
  fn = pl.kernel(
      _sc_body,
      out_type=jax.ShapeDtypeStruct((BATCH, PAD), jnp.float32),
      mesh=mesh,
      compiler_params=pltpu.CompilerParams(
          needs_layout_passes=False, use_tc_tiling_on_sc=True),
      scratch_types=[
          pltpu.VMEM((ROWS, IMG), jnp.float32),
          pltpu.VMEM((ROWS, IMG), jnp.float32),
          pltpu.VMEM((LANES, NBINS), jnp.float32),
          pltpu.VMEM((PAD,), jnp.float32),
          pltpu.SemaphoreType.DMA,
          pltpu.SemaphoreType.DMA,
      ],
  )
  return fn(x)


def _tc_hist_body(x_ref, out_ref):
  k = pl.program_id(1)
  rows = []
  for c in range(CHANNELS):
    xc = x_ref[0, c]
    rows.append(jnp.full((IMG,), jnp.float32(RB)))
    rows.extend(jnp.sum((xc >= (i / 16.0)).astype(jnp.float32), axis=0)
                for i in range(1, NBINS))
  blk = jnp.stack(rows).reshape(1, FEAT, IMG)

  @pl.when(k == 0)
  def _():
    out_ref[...] = blk

  @pl.when(k != 0)
  def _():
    out_ref[...] += blk


def _tc_counts(x):
  return pl.pallas_call(
      _tc_hist_body,
      grid=(BATCH, K_TC),
      in_specs=[pl.BlockSpec((1, CHANNELS, RB, IMG),
                             lambda i, k: (i, 0, k + OFF_B, 0))],
      out_specs=pl.BlockSpec((1, FEAT, IMG), lambda i, k: (i, 0, 0)),
      out_shape=jax.ShapeDtypeStruct((BATCH, FEAT, IMG), jnp.float32),
  )(x)


def _fc_body(a_ref, c_ref, wp_ref, w_ref, bias_ref, o_ref):
  tails = jnp.sum(c_ref[...], axis=2)
  nxt = jnp.concatenate(
      [tails[:, 1:], jnp.zeros((BATCH, 1), jnp.float32)], axis=1)
  mask = (lax.broadcasted_iota(jnp.int32, (1, FEAT), 1) % NBINS
          ) != (NBINS - 1)
  cnt_tc = tails - jnp.where(mask, nxt, 0.0)
  scale = jnp.float32(1.0 / PLANE)
  o = jnp.dot(a_ref[...] * scale, wp_ref[...],
              preferred_element_type=jnp.float32)
  o += jnp.dot(cnt_tc * scale, w_ref[...],
               preferred_element_type=jnp.float32)
  o_ref[...] = jnp.maximum(o + bias_ref[...][None, :], 0.0)


def _fc(cnt_sc, cnt_tc, w_pad, w, bias):
  return pl.pallas_call(
      _fc_body,
      out_shape=jax.ShapeDtypeStruct((BATCH, OUT_DIM), jnp.float32),
  )(cnt_sc, cnt_tc, w_pad, w, bias)


@jax.jit
def kernel(x, W, b):
  cnt_sc = _sc_counts(x)
  cnt_tc = _tc_counts(x)
  w_pad = jnp.zeros((PAD, OUT_DIM), jnp.float32).at[:FEAT].set(W)
  return _fc(cnt_sc, cnt_tc, w_pad, W, b)
